# baseline (speedup 1.0000x reference)
import math

import jax
import jax.numpy as jnp
from jax.experimental import pallas as pl

TIME_STEPS = 256
NUM_NODES = 128
EMBED_DIM = 16
CHANNELS = 144
CB = 16


def _body(wt_ref, out_ref):
    j = pl.program_id(1)

    @pl.when(j < 8)
    def _time():
        c0 = j * CB
        ci = jax.lax.broadcasted_iota(jnp.int32, (CB, TIME_STEPS), 0) + c0
        li = jax.lax.broadcasted_iota(jnp.int32, (CB, TIME_STEPS), 1).astype(
            jnp.float32
        )
        half = (ci >> 1).astype(jnp.float32)
        inv_freq = jnp.exp(half * (-2.0 * math.log(10000.0) / 128.0))
        ang = li * inv_freq
        pe = jnp.where((ci & 1) == 0, jnp.sin(ang), jnp.cos(ang))
        out_ref[0] = jnp.broadcast_to(pe[:, None, :], (CB, NUM_NODES, TIME_STEPS))

    @pl.when(j == 8)
    def _embed():
        wt = wt_ref[...]
        out_ref[0] = jnp.broadcast_to(
            wt[:, :, None], (EMBED_DIM, NUM_NODES, TIME_STEPS)
        )


def kernel(cond_mask, embed_weight):
    B = cond_mask.shape[0]
    wt = embed_weight.T
    return pl.pallas_call(
        _body,
        grid=(B, 9),
        in_specs=[
            pl.BlockSpec((EMBED_DIM, NUM_NODES), lambda b, j: (0, 0)),
        ],
        out_specs=pl.BlockSpec(
            (1, CB, NUM_NODES, TIME_STEPS), lambda b, j: (b, j, 0, 0)
        ),
        out_shape=jax.ShapeDtypeStruct(
            (B, CHANNELS, NUM_NODES, TIME_STEPS), jnp.float32
        ),
    )(wt)


# speedup vs baseline: 1.1381x; 1.1381x over previous
import math

import jax
import jax.numpy as jnp
from jax.experimental import pallas as pl
from jax.experimental.pallas import tpu as pltpu

TIME_STEPS = 256
NUM_NODES = 128
EMBED_DIM = 16
CHANNELS = 144


def _body(wt_ref, out_ref, buf, sems):
    b = pl.program_id(0)
    nb = pl.num_programs(0)

    @pl.when(b == 0)
    def _fill():
        ci = jax.lax.broadcasted_iota(jnp.int32, (128, TIME_STEPS), 0)
        li = jax.lax.broadcasted_iota(jnp.int32, (128, TIME_STEPS), 1).astype(
            jnp.float32
        )
        half = (ci >> 1).astype(jnp.float32)
        inv_freq = jnp.exp(half * (-2.0 * math.log(10000.0) / 128.0))
        ang = li * inv_freq
        pe = jnp.where((ci & 1) == 0, jnp.sin(ang), jnp.cos(ang))
        buf[:128] = jnp.broadcast_to(pe[:, None, :], (128, NUM_NODES, TIME_STEPS))
        wt = wt_ref[...]
        buf[128:] = jnp.broadcast_to(
            wt[:, :, None], (EMBED_DIM, NUM_NODES, TIME_STEPS)
        )

    pltpu.make_async_copy(buf, out_ref.at[b], sems.at[b % 2]).start()

    @pl.when(b > 0)
    def _wait_prev():
        pltpu.make_async_copy(buf, out_ref.at[b - 1], sems.at[(b - 1) % 2]).wait()

    @pl.when(b == nb - 1)
    def _wait_last():
        pltpu.make_async_copy(buf, out_ref.at[b], sems.at[b % 2]).wait()


def kernel(cond_mask, embed_weight):
    B = cond_mask.shape[0]
    wt = embed_weight.T
    return pl.pallas_call(
        _body,
        grid=(B,),
        in_specs=[
            pl.BlockSpec((EMBED_DIM, NUM_NODES), lambda b: (0, 0)),
        ],
        out_specs=pl.BlockSpec(memory_space=pl.ANY),
        out_shape=jax.ShapeDtypeStruct(
            (B, CHANNELS, NUM_NODES, TIME_STEPS), jnp.float32
        ),
        scratch_shapes=[
            pltpu.VMEM((CHANNELS, NUM_NODES, TIME_STEPS), jnp.float32),
            pltpu.SemaphoreType.DMA((2,)),
        ],
    )(wt)
